# Initial kernel scaffold; baseline (speedup 1.0000x reference)
import functools

import numpy as np
import jax
import jax.numpy as jnp
from jax import lax
from jax.experimental import pallas as pl
from jax.experimental.pallas import tpu as pltpu
from jax.experimental.pallas import tpu_sc as plsc

SEQ_LEN = 200
BATCH = 4096
VOCAB = 100000
DIM = 64

_info = plsc.get_sparse_core_info()
NC = _info.num_cores
NS = _info.num_subcores
NW = NC * NS
TOTAL = BATCH * SEQ_LEN
BPW = TOTAL // NW
CHUNK = 128
NCHUNK = BPW // CHUNK


def _pe_last_position() -> np.ndarray:
    pos = SEQ_LEN - 1
    pe = np.zeros(DIM)
    for i in range(DIM):
        if i % 2 == 0:
            pe[i] = np.sin(pos / 10000 ** (i / DIM))
        else:
            pe[i] = np.cos(pos / 10000 ** ((i - 1) / DIM))
    return pe.astype(np.float32)


_PE8 = jnp.asarray(np.tile(_pe_last_position()[None, :], (8, 1)))


def _pe_add_body(t_ref, pe_ref, o_ref):
    o_ref[...] = t_ref[...] + pe_ref[0:1, :]


def _add_pe(table):
    nblk = 10
    rows = VOCAB // nblk
    return pl.pallas_call(
        _pe_add_body,
        grid=(nblk,),
        in_specs=[
            pl.BlockSpec((rows, DIM), lambda i: (i, 0)),
            pl.BlockSpec((8, DIM), lambda i: (0, 0)),
        ],
        out_specs=pl.BlockSpec((rows, DIM), lambda i: (i, 0)),
        out_shape=jax.ShapeDtypeStruct((VOCAB, DIM), jnp.float32),
    )(table, _PE8)


@functools.partial(
    pl.kernel,
    mesh=plsc.VectorSubcoreMesh(core_axis_name="c", subcore_axis_name="s"),
    out_type=jax.ShapeDtypeStruct((TOTAL, DIM), jnp.float32),
    scratch_types=[
        pltpu.VMEM((NCHUNK, CHUNK), jnp.int32),
        pltpu.VMEM((CHUNK, DIM), jnp.float32),
        pltpu.SemaphoreType.DMA,
    ],
)
def _gather(table_hbm, idx_hbm, out_hbm, idx_v, rows_v, sem):
    wid = lax.axis_index("s") * NC + lax.axis_index("c")
    base = wid * BPW
    pltpu.sync_copy(idx_hbm.at[wid], idx_v)

    def chunk(j, carry):
        pltpu.async_copy(table_hbm.at[idx_v.at[j]], rows_v, sem).wait()
        pltpu.sync_copy(rows_v, out_hbm.at[pl.ds(base + j * CHUNK, CHUNK)])
        return carry

    lax.fori_loop(0, NCHUNK, chunk, 0)


def kernel(inputs, table):
    table_pe = _add_pe(table)
    idx = inputs.reshape(NW, NCHUNK, CHUNK)
    out = _gather(table_pe, idx)
    return out.reshape(BATCH, SEQ_LEN, DIM)


# speedup vs baseline: 3.3449x; 3.3449x over previous
import functools

import numpy as np
import jax
import jax.numpy as jnp
from jax import lax
from jax.experimental import pallas as pl
from jax.experimental.pallas import tpu as pltpu
from jax.experimental.pallas import tpu_sc as plsc

SEQ_LEN = 200
BATCH = 4096
VOCAB = 100000
DIM = 64

_info = plsc.get_sparse_core_info()
NC = _info.num_cores
NS = _info.num_subcores
NW = NC * NS
TOTAL = BATCH * SEQ_LEN
BPW = TOTAL // NW
CHUNK = 128
NCHUNK = BPW // CHUNK


def _pe_last_position() -> np.ndarray:
    pos = SEQ_LEN - 1
    pe = np.zeros(DIM)
    for i in range(DIM):
        if i % 2 == 0:
            pe[i] = np.sin(pos / 10000 ** (i / DIM))
        else:
            pe[i] = np.cos(pos / 10000 ** ((i - 1) / DIM))
    return pe.astype(np.float32)


_PE8 = np.tile(_pe_last_position()[None, :], (8, 1))


def _pe_add_body(t_ref, pe_ref, o_ref):
    o_ref[...] = t_ref[...] + pe_ref[0:1, :]


def _add_pe(table):
    nblk = 10
    rows = VOCAB // nblk
    return pl.pallas_call(
        _pe_add_body,
        grid=(nblk,),
        in_specs=[
            pl.BlockSpec((rows, DIM), lambda i: (i, 0)),
            pl.BlockSpec((8, DIM), lambda i: (0, 0)),
        ],
        out_specs=pl.BlockSpec((rows, DIM), lambda i: (i, 0)),
        out_shape=jax.ShapeDtypeStruct((VOCAB, DIM), jnp.float32),
    )(table, jnp.asarray(_PE8))


@functools.partial(
    pl.kernel,
    mesh=plsc.VectorSubcoreMesh(core_axis_name="c", subcore_axis_name="s"),
    out_type=jax.ShapeDtypeStruct((TOTAL, DIM), jnp.float32),
    scratch_types=[
        pltpu.VMEM((NCHUNK, CHUNK), jnp.int32),
        pltpu.VMEM((CHUNK, DIM), jnp.float32),
        pltpu.SemaphoreType.DMA,
    ],
    compiler_params=pltpu.CompilerParams(use_tc_tiling_on_sc=False),
)
def _gather(table_hbm, idx_hbm, out_hbm, idx_v, rows_v, sem):
    wid = lax.axis_index("s") * NC + lax.axis_index("c")
    base = wid * BPW
    pltpu.sync_copy(idx_hbm.at[wid], idx_v)

    def chunk(j, carry):
        pltpu.async_copy(table_hbm.at[idx_v.at[j]], rows_v, sem).wait()
        pltpu.sync_copy(rows_v, out_hbm.at[pl.ds(base + j * CHUNK, CHUNK)])
        return carry

    lax.fori_loop(0, NCHUNK, chunk, 0)


def kernel(inputs, table):
    table_pe = _add_pe(table)
    idx = inputs.reshape(NW, NCHUNK, CHUNK)
    out = _gather(table_pe, idx)
    return out.reshape(BATCH, SEQ_LEN, DIM)


# speedup vs baseline: 3.9687x; 1.1865x over previous
import functools

import numpy as np
import jax
import jax.numpy as jnp
from jax import lax
from jax.experimental import pallas as pl
from jax.experimental.pallas import tpu as pltpu
from jax.experimental.pallas import tpu_sc as plsc

SEQ_LEN = 200
BATCH = 4096
VOCAB = 100000
DIM = 64

_info = plsc.get_sparse_core_info()
NC = _info.num_cores
NS = _info.num_subcores
NW = NC * NS
TOTAL = BATCH * SEQ_LEN
BPW = TOTAL // NW
CHUNK = 128
NCHUNK = BPW // CHUNK
GCHUNKS = 5
ROWS_G = GCHUNKS * CHUNK
NGROUP = NCHUNK // GCHUNKS


def _pe_last_position() -> np.ndarray:
    pos = SEQ_LEN - 1
    pe = np.zeros(DIM)
    for i in range(DIM):
        if i % 2 == 0:
            pe[i] = np.sin(pos / 10000 ** (i / DIM))
        else:
            pe[i] = np.cos(pos / 10000 ** ((i - 1) / DIM))
    return pe.astype(np.float32)


_PE8 = np.tile(_pe_last_position()[None, :], (8, 1))


def _pe_add_body(t_ref, pe_ref, o_ref):
    o_ref[...] = t_ref[...] + pe_ref[0:1, :]


def _add_pe(table):
    nblk = 10
    rows = VOCAB // nblk
    return pl.pallas_call(
        _pe_add_body,
        grid=(nblk,),
        in_specs=[
            pl.BlockSpec((rows, DIM), lambda i: (i, 0)),
            pl.BlockSpec((8, DIM), lambda i: (0, 0)),
        ],
        out_specs=pl.BlockSpec((rows, DIM), lambda i: (i, 0)),
        out_shape=jax.ShapeDtypeStruct((VOCAB, DIM), jnp.float32),
    )(table, jnp.asarray(_PE8))


def _gathers_desc(table_hbm, idx_v, buf, sem, g):
    for i in range(GCHUNKS):
        yield pltpu.make_async_copy(
            table_hbm.at[idx_v.at[g * GCHUNKS + i]],
            buf.at[pl.ds(i * CHUNK, CHUNK)],
            sem,
        )


def _copyout_desc(out_hbm, buf, sem, base, g):
    return pltpu.make_async_copy(
        buf, out_hbm.at[pl.ds(base + g * ROWS_G, ROWS_G)], sem
    )


@functools.partial(
    pl.kernel,
    mesh=plsc.VectorSubcoreMesh(core_axis_name="c", subcore_axis_name="s"),
    out_type=jax.ShapeDtypeStruct((TOTAL, DIM), jnp.float32),
    scratch_types=[
        pltpu.VMEM((NCHUNK, CHUNK), jnp.int32),
        pltpu.VMEM((ROWS_G, DIM), jnp.float32),
        pltpu.VMEM((ROWS_G, DIM), jnp.float32),
        pltpu.SemaphoreType.DMA,
        pltpu.SemaphoreType.DMA,
        pltpu.SemaphoreType.DMA,
        pltpu.SemaphoreType.DMA,
    ],
    compiler_params=pltpu.CompilerParams(use_tc_tiling_on_sc=False),
)
def _gather(table_hbm, idx_hbm, out_hbm, idx_v, buf_a, buf_b, sga, sgb, soa, sob):
    wid = lax.axis_index("s") * NC + lax.axis_index("c")
    base = wid * BPW
    pltpu.sync_copy(idx_hbm.at[wid], idx_v)

    for d in _gathers_desc(table_hbm, idx_v, buf_a, sga, 0):
        d.start()

    def step(t, carry):
        ga = 2 * t
        gb = 2 * t + 1

        @pl.when(t > 0)
        def _():
            _copyout_desc(out_hbm, buf_b, sob, base, gb - 2).wait()

        for d in _gathers_desc(table_hbm, idx_v, buf_b, sgb, gb):
            d.start()
        for d in _gathers_desc(table_hbm, idx_v, buf_a, sga, ga):
            d.wait()
        _copyout_desc(out_hbm, buf_a, soa, base, ga).start()
        _copyout_desc(out_hbm, buf_a, soa, base, ga).wait()

        @pl.when(t < NGROUP // 2 - 1)
        def _():
            for d in _gathers_desc(table_hbm, idx_v, buf_a, sga, ga + 2):
                d.start()

        for d in _gathers_desc(table_hbm, idx_v, buf_b, sgb, gb):
            d.wait()
        _copyout_desc(out_hbm, buf_b, sob, base, gb).start()
        return carry

    lax.fori_loop(0, NGROUP // 2, step, 0)
    _copyout_desc(out_hbm, buf_b, sob, base, NGROUP - 1).wait()


def kernel(inputs, table):
    table_pe = _add_pe(table)
    idx = inputs.reshape(NW, NCHUNK, CHUNK)
    out = _gather(table_pe, idx)
    return out.reshape(BATCH, SEQ_LEN, DIM)


# speedup vs baseline: 3.9708x; 1.0005x over previous
import functools

import numpy as np
import jax
import jax.numpy as jnp
from jax import lax
from jax.experimental import pallas as pl
from jax.experimental.pallas import tpu as pltpu
from jax.experimental.pallas import tpu_sc as plsc

SEQ_LEN = 200
BATCH = 4096
VOCAB = 100000
DIM = 64

_info = plsc.get_sparse_core_info()
NC = _info.num_cores
NS = _info.num_subcores
NW = NC * NS
TOTAL = BATCH * SEQ_LEN
BPW = TOTAL // NW
BATCH_W = BATCH // NW
NB = 2
NGROUP = BATCH_W // NB
SPLITS = ((0, 128), (128, 72))


def _pe_last_position() -> np.ndarray:
    pos = SEQ_LEN - 1
    pe = np.zeros(DIM)
    for i in range(DIM):
        if i % 2 == 0:
            pe[i] = np.sin(pos / 10000 ** (i / DIM))
        else:
            pe[i] = np.cos(pos / 10000 ** ((i - 1) / DIM))
    return pe.astype(np.float32)


_PE8 = np.tile(_pe_last_position()[None, :], (8, 1))


def _pe_add_body(t_ref, pe_ref, o_ref):
    o_ref[...] = t_ref[...] + pe_ref[0:1, :]


def _add_pe(table):
    nblk = 10
    rows = VOCAB // nblk
    return pl.pallas_call(
        _pe_add_body,
        grid=(nblk,),
        in_specs=[
            pl.BlockSpec((rows, DIM), lambda i: (i, 0)),
            pl.BlockSpec((8, DIM), lambda i: (0, 0)),
        ],
        out_specs=pl.BlockSpec((rows, DIM), lambda i: (i, 0)),
        out_shape=jax.ShapeDtypeStruct((VOCAB, DIM), jnp.float32),
    )(table, jnp.asarray(_PE8))


def _gathers_desc(table_hbm, idx_v, buf, sem, g):
    for bb in range(NB):
        for off, n in SPLITS:
            yield pltpu.make_async_copy(
                table_hbm.at[idx_v.at[NB * g + bb, pl.ds(off, n)]],
                buf.at[bb, pl.ds(off, n)],
                sem,
            )


def _copyout_desc(out_hbm, buf, sem, bbase, g):
    return pltpu.make_async_copy(buf, out_hbm.at[pl.ds(bbase + g * NB, NB)], sem)


@functools.partial(
    pl.kernel,
    mesh=plsc.VectorSubcoreMesh(core_axis_name="c", subcore_axis_name="s"),
    out_type=jax.ShapeDtypeStruct((BATCH, SEQ_LEN, DIM), jnp.float32),
    scratch_types=[
        pltpu.VMEM((BATCH_W, SEQ_LEN), jnp.int32),
        pltpu.VMEM((NB, SEQ_LEN, DIM), jnp.float32),
        pltpu.VMEM((NB, SEQ_LEN, DIM), jnp.float32),
        pltpu.SemaphoreType.DMA,
        pltpu.SemaphoreType.DMA,
        pltpu.SemaphoreType.DMA,
        pltpu.SemaphoreType.DMA,
    ],
    compiler_params=pltpu.CompilerParams(use_tc_tiling_on_sc=False),
)
def _gather(table_hbm, idx_hbm, out_hbm, idx_v, buf_a, buf_b, sga, sgb, soa, sob):
    wid = lax.axis_index("s") * NC + lax.axis_index("c")
    bbase = wid * BATCH_W
    pltpu.sync_copy(idx_hbm.at[pl.ds(bbase, BATCH_W)], idx_v)

    def fire_gathers(buf, sem, g):
        for d in _gathers_desc(table_hbm, idx_v, buf, sem, g):
            d.start()

    def wait_gathers(buf, sem, g):
        for d in _gathers_desc(table_hbm, idx_v, buf, sem, g):
            d.wait()

    fire_gathers(buf_a, sga, 0)

    def step(t, carry):
        ga = 2 * t
        gb = 2 * t + 1

        @pl.when(t > 0)
        def _():
            _copyout_desc(out_hbm, buf_b, sob, bbase, gb - 2).wait()

        fire_gathers(buf_b, sgb, gb)
        wait_gathers(buf_a, sga, ga)
        _copyout_desc(out_hbm, buf_a, soa, bbase, ga).start()
        _copyout_desc(out_hbm, buf_a, soa, bbase, ga).wait()

        @pl.when(t < NGROUP // 2 - 1)
        def _():
            fire_gathers(buf_a, sga, ga + 2)

        wait_gathers(buf_b, sgb, gb)
        _copyout_desc(out_hbm, buf_b, sob, bbase, gb).start()
        return carry

    lax.fori_loop(0, NGROUP // 2, step, 0)
    _copyout_desc(out_hbm, buf_b, sob, bbase, NGROUP - 1).wait()


def kernel(inputs, table):
    table_pe = _add_pe(table)
    return _gather(table_pe, inputs)


# speedup vs baseline: 5.3691x; 1.3522x over previous
import functools

import numpy as np
import jax
import jax.numpy as jnp
from jax import lax
from jax.experimental import pallas as pl
from jax.experimental.pallas import tpu as pltpu
from jax.experimental.pallas import tpu_sc as plsc

SEQ_LEN = 200
BATCH = 4096
VOCAB = 100000
DIM = 64

_info = plsc.get_sparse_core_info()
NC = _info.num_cores
NS = _info.num_subcores
NW = NC * NS
BATCH_W = BATCH // NW
SPLITS = ((0, 128), (128, 72))


def _pe_last_position() -> np.ndarray:
    pos = SEQ_LEN - 1
    pe = np.zeros(DIM)
    for i in range(DIM):
        if i % 2 == 0:
            pe[i] = np.sin(pos / 10000 ** (i / DIM))
        else:
            pe[i] = np.cos(pos / 10000 ** ((i - 1) / DIM))
    return pe.astype(np.float32)


_PE8 = np.tile(_pe_last_position()[None, :], (8, 1))


def _pe_add_body(t_ref, pe_ref, o_ref):
    x = t_ref[...] + pe_ref[0:1, :]
    o_ref[:, 0:DIM] = x
    o_ref[:, DIM:2 * DIM] = x


def _add_pe_dup(table):
    nblk = 10
    rows = VOCAB // nblk
    return pl.pallas_call(
        _pe_add_body,
        grid=(nblk,),
        in_specs=[
            pl.BlockSpec((rows, DIM), lambda i: (i, 0)),
            pl.BlockSpec((8, DIM), lambda i: (0, 0)),
        ],
        out_specs=pl.BlockSpec((rows, 2 * DIM), lambda i: (i, 0)),
        out_shape=jax.ShapeDtypeStruct((VOCAB, 2 * DIM), jnp.float32),
    )(table, jnp.asarray(_PE8))


def _gathers_desc(table_hbm, idx_v, buf, sem, lb):
    for h, (off, n) in enumerate(SPLITS):
        yield pltpu.make_async_copy(
            table_hbm.at[idx_v.at[2 * lb + h, pl.ds(0, n)]],
            buf.at[pl.ds(off, n)],
            sem,
        )


def _copyout_desc(out_hbm, buf, sem, bbase, lb):
    return pltpu.make_async_copy(buf, out_hbm.at[bbase + lb], sem)


@functools.partial(
    pl.kernel,
    mesh=plsc.VectorSubcoreMesh(core_axis_name="c", subcore_axis_name="s"),
    out_type=jax.ShapeDtypeStruct((BATCH, SEQ_LEN, 2 * DIM), jnp.float32),
    scratch_types=[
        pltpu.VMEM((2 * BATCH_W, 128), jnp.int32),
        pltpu.VMEM((SEQ_LEN, 2 * DIM), jnp.float32),
        pltpu.VMEM((SEQ_LEN, 2 * DIM), jnp.float32),
        pltpu.SemaphoreType.DMA,
        pltpu.SemaphoreType.DMA,
        pltpu.SemaphoreType.DMA,
        pltpu.SemaphoreType.DMA,
    ],
    compiler_params=pltpu.CompilerParams(use_tc_tiling_on_sc=True),
)
def _gather(table_hbm, idx_hbm, out_hbm, idx_v, buf_a, buf_b, sga, sgb, soa, sob):
    wid = lax.axis_index("s") * NC + lax.axis_index("c")
    bbase = wid * BATCH_W
    pltpu.sync_copy(idx_hbm.at[pl.ds(wid * 2 * BATCH_W, 2 * BATCH_W)], idx_v)

    def fire_gathers(buf, sem, lb):
        for d in _gathers_desc(table_hbm, idx_v, buf, sem, lb):
            d.start()

    def wait_gathers(buf, sem, lb):
        for d in _gathers_desc(table_hbm, idx_v, buf, sem, lb):
            d.wait()

    fire_gathers(buf_a, sga, 0)

    def step(t, carry):
        ga = 2 * t
        gb = 2 * t + 1

        @pl.when(t > 0)
        def _():
            _copyout_desc(out_hbm, buf_b, sob, bbase, gb - 2).wait()

        fire_gathers(buf_b, sgb, gb)
        wait_gathers(buf_a, sga, ga)
        _copyout_desc(out_hbm, buf_a, soa, bbase, ga).start()
        _copyout_desc(out_hbm, buf_a, soa, bbase, ga).wait()

        @pl.when(t < BATCH_W // 2 - 1)
        def _():
            fire_gathers(buf_a, sga, ga + 2)

        wait_gathers(buf_b, sgb, gb)
        _copyout_desc(out_hbm, buf_b, sob, bbase, gb).start()
        return carry

    lax.fori_loop(0, BATCH_W // 2, step, 0)
    _copyout_desc(out_hbm, buf_b, sob, bbase, BATCH_W - 1).wait()


def kernel(inputs, table):
    table_dup = _add_pe_dup(table)
    idx = jnp.pad(inputs, ((0, 0), (0, 256 - SEQ_LEN))).reshape(2 * BATCH, 128)
    return _gather(table_dup, idx)[:, :, :DIM]
